# Initial kernel scaffold; baseline (speedup 1.0000x reference)
import jax
import jax.numpy as jnp
from jax.experimental import pallas as pl


def kernel(x, edge_index, edge_type, edge_norm, weights):
    raise NotImplementedError("write your pallas kernel here")


# speedup vs baseline: 3.2634x; 3.2634x over previous
import dataclasses
import functools

import jax
import jax.numpy as jnp
from jax import lax
from jax.experimental import pallas as pl
from jax.experimental.pallas import tpu as pltpu
from jax.experimental.pallas import tpu_sc as plsc

N_NODES = 10000
E_EDGES = 320000
D = 128
R = 8

NC = 2
NS = 16
NW = NC * NS
LANES = 16

EDGES_PER_TILE = E_EDGES // NW
CHUNK = 80
NCHUNK = EDGES_PER_TILE // CHUNK
N_PAD = 10240
ROWS_PER_TILE = N_PAD // NS


_BN = 2000


def _mm_body(x_ref, w_ref, y_ref):
    y_ref[0] = lax.dot_general(
        x_ref[...], w_ref[0], (((1,), (0,)), ((), ())),
        preferred_element_type=jnp.float32,
        precision=lax.Precision.HIGHEST,
    )


_mm = pl.pallas_call(
    _mm_body,
    grid=(R, N_NODES // _BN),
    in_specs=[
        pl.BlockSpec((_BN, D), lambda r, nb: (nb, 0)),
        pl.BlockSpec((1, D, D), lambda r, nb: (r, 0, 0)),
    ],
    out_specs=pl.BlockSpec((1, _BN, D), lambda r, nb: (r, nb, 0)),
    out_shape=jax.ShapeDtypeStruct((R, N_NODES, D), jnp.float32),
)


_mesh = plsc.VectorSubcoreMesh(core_axis_name="c", subcore_axis_name="s")

_sc_params = pltpu.CompilerParams()
if "needs_layout_passes" in pltpu.CompilerParams.__dataclass_fields__:
    _sc_params = dataclasses.replace(_sc_params, needs_layout_passes=False)


@functools.partial(
    pl.kernel,
    out_type=jax.ShapeDtypeStruct((NC, N_PAD, D), jnp.float32),
    mesh=_mesh,
    compiler_params=_sc_params,
    scratch_types=[
        pltpu.VMEM((CHUNK,), jnp.int32),
        pltpu.VMEM((CHUNK,), jnp.int32),
        pltpu.VMEM((CHUNK,), jnp.int32),
        pltpu.VMEM((CHUNK,), jnp.int32),
        pltpu.VMEM((CHUNK,), jnp.float32),
        pltpu.VMEM((CHUNK, D), jnp.float32),
        pltpu.VMEM_SHARED((N_PAD, D), jnp.float32),
        pltpu.SemaphoreType.DMA,
    ],
)
def _sc_edges(y_hbm, src_hbm, dst_hbm, type_hbm, norm_hbm, out_hbm,
              gidx_v, src_v, type_v, dst_v, norm_v, rows_v, acc_sh, sem):
    c = lax.axis_index("c")
    s = lax.axis_index("s")
    wid = c * NS + s

    zero16 = jnp.zeros((LANES,), jnp.float32)

    @pl.loop(0, CHUNK)
    def _zero_rows(i):
        for j in range(D // LANES):
            rows_v[i, pl.ds(j * LANES, LANES)] = zero16

    row0 = s * ROWS_PER_TILE
    for i in range(ROWS_PER_TILE // CHUNK):
        pltpu.sync_copy(rows_v.at[pl.ds(0, CHUNK)],
                        acc_sh.at[pl.ds(row0 + i * CHUNK, CHUNK)])
    plsc.subcore_barrier()

    e0 = wid * EDGES_PER_TILE

    @pl.loop(0, NCHUNK)
    def _edge_chunk(k):
        base = e0 + k * CHUNK
        pltpu.sync_copy(src_hbm.at[pl.ds(base, CHUNK)], src_v)
        pltpu.sync_copy(type_hbm.at[pl.ds(base, CHUNK)], type_v)
        pltpu.sync_copy(dst_hbm.at[pl.ds(base, CHUNK)], dst_v)
        pltpu.sync_copy(norm_hbm.at[pl.ds(base, CHUNK)], norm_v)

        for i in range(CHUNK // LANES):
            sl = pl.ds(i * LANES, LANES)
            gidx_v[sl] = type_v[sl] * N_NODES + src_v[sl]

        pltpu.async_copy(y_hbm.at[gidx_v], rows_v, sem).wait()

        @pl.loop(0, CHUNK)
        def _scale(e):
            nvec = plsc.load_gather(norm_v, [jnp.zeros((LANES,), jnp.int32) + e])
            for j in range(D // LANES):
                sl = pl.ds(j * LANES, LANES)
                rows_v[e, sl] = rows_v[e, sl] * nvec

        pltpu.sync_copy(rows_v, acc_sh.at[dst_v], add=True)

    plsc.subcore_barrier()
    pltpu.sync_copy(acc_sh.at[pl.ds(row0, ROWS_PER_TILE)],
                    out_hbm.at[c, pl.ds(row0, ROWS_PER_TILE)])


_BA = 2000


def _add_body(a_ref, b_ref, o_ref):
    o_ref[...] = a_ref[...] + b_ref[...]


_add = pl.pallas_call(
    _add_body,
    grid=(N_NODES // _BA,),

    in_specs=[
        pl.BlockSpec((_BA, D), lambda i: (i, 0)),
        pl.BlockSpec((_BA, D), lambda i: (i, 0)),
    ],
    out_specs=pl.BlockSpec((_BA, D), lambda i: (i, 0)),
    out_shape=jax.ShapeDtypeStruct((N_NODES, D), jnp.float32),
)


def kernel(x, edge_index, edge_type, edge_norm, weights):
    src = edge_index[0]
    dst = edge_index[1]
    y = _mm(x, weights).reshape(R * N_NODES, D)
    partial = _sc_edges(y, src, dst, edge_type, edge_norm)
    return _add(partial[0], partial[1])


# speedup vs baseline: 6.3630x; 1.9498x over previous
import dataclasses
import functools

import jax
import jax.numpy as jnp
from jax import lax
from jax.experimental import pallas as pl
from jax.experimental.pallas import tpu as pltpu
from jax.experimental.pallas import tpu_sc as plsc

N_NODES = 10000
E_EDGES = 320000
D = 128
R = 8

NC = 2
NS = 16
NW = NC * NS
LANES = 16

EDGES_PER_TILE = E_EDGES // NW
CHUNK = 40
BPC = 50
NBLOCK = EDGES_PER_TILE // (BPC * CHUNK)
N_PAD = 10240
ROWS_PER_TILE = N_PAD // NS


_BN = 2000


def _mm_body(x_ref, w_ref, y_ref):
    y_ref[0] = lax.dot_general(
        x_ref[...], w_ref[0], (((1,), (0,)), ((), ())),
        preferred_element_type=jnp.float32,
        precision=lax.Precision.HIGHEST,
    )


_mm = pl.pallas_call(
    _mm_body,
    grid=(R, N_NODES // _BN),
    in_specs=[
        pl.BlockSpec((_BN, D), lambda r, nb: (nb, 0)),
        pl.BlockSpec((1, D, D), lambda r, nb: (r, 0, 0)),
    ],
    out_specs=pl.BlockSpec((1, _BN, D), lambda r, nb: (r, nb, 0)),
    out_shape=jax.ShapeDtypeStruct((R, N_NODES, D), jnp.float32),
)


_mesh = plsc.VectorSubcoreMesh(core_axis_name="c", subcore_axis_name="s")

_sc_params = pltpu.CompilerParams()
if "needs_layout_passes" in pltpu.CompilerParams.__dataclass_fields__:
    _sc_params = dataclasses.replace(_sc_params, needs_layout_passes=False)


NBUF = 2


@functools.partial(
    pl.kernel,
    out_type=jax.ShapeDtypeStruct((NC, N_PAD, D), jnp.float32),
    mesh=_mesh,
    compiler_params=_sc_params,
    scratch_types=[
        pltpu.VMEM((BPC, CHUNK), jnp.int32),
        pltpu.VMEM((BPC, CHUNK), jnp.int32),
        pltpu.VMEM((BPC * CHUNK,), jnp.float32),
        *[pltpu.VMEM((CHUNK, D), jnp.float32) for _ in range(NBUF)],
        pltpu.VMEM_SHARED((N_PAD, D), jnp.float32),
        *[pltpu.SemaphoreType.DMA for _ in range(2 * NBUF)],
    ],
)
def _sc_edges(y_hbm, gidx_hbm, dst_hbm, norm_hbm, out_hbm,
              gidx_v, dst_v, norm_v, *bufs_and_sems):
    rows = bufs_and_sems[:NBUF]
    acc_sh = bufs_and_sems[NBUF]
    g_sems = bufs_and_sems[NBUF + 1:NBUF + 1 + NBUF]
    s_sems = bufs_and_sems[NBUF + 1 + NBUF:]
    c = lax.axis_index("c")
    s = lax.axis_index("s")
    wid = c * NS + s

    zero16 = jnp.zeros((LANES,), jnp.float32)

    @pl.loop(0, CHUNK)
    def _zero_rows(i):
        for j in range(D // LANES):
            rows[0][i, pl.ds(j * LANES, LANES)] = zero16

    row0 = s * ROWS_PER_TILE
    for i in range(ROWS_PER_TILE // CHUNK):
        pltpu.sync_copy(rows[0].at[pl.ds(0, CHUNK)],
                        acc_sh.at[pl.ds(row0 + i * CHUNK, CHUNK)])
    plsc.subcore_barrier()

    @pl.loop(0, NBLOCK)
    def _block(blk):
        pltpu.sync_copy(gidx_hbm.at[wid, blk], gidx_v)
        pltpu.sync_copy(dst_hbm.at[wid, blk], dst_v)
        pltpu.sync_copy(norm_hbm.at[wid, blk], norm_v)

        for b in range(NBUF):
            pltpu.async_copy(y_hbm.at[gidx_v.at[b]], rows[b], g_sems[b])

        @pl.loop(0, BPC, step=NBUF)
        def _group(k0):
            for b in range(NBUF):
                k = k0 + b
                pltpu.make_async_copy(y_hbm.at[gidx_v.at[0]], rows[b],
                                      g_sems[b]).wait()

                @plsc.parallel_loop(0, CHUNK, unroll=2)
                def _scale(e):
                    nvec = plsc.load_gather(
                        norm_v,
                        [jnp.zeros((LANES,), jnp.int32) + (k * CHUNK + e)])
                    for j in range(D // LANES):
                        sl = pl.ds(j * LANES, LANES)
                        rows[b][e, sl] = rows[b][e, sl] * nvec

                pltpu.async_copy(rows[b], acc_sh.at[dst_v.at[k]], s_sems[b],
                                 add=True)
            for b in range(NBUF):
                pltpu.make_async_copy(rows[b], acc_sh.at[dst_v.at[0]],
                                      s_sems[b]).wait()
                nk = k0 + NBUF + b

                @pl.when(nk < BPC)
                def _prefetch():
                    pltpu.async_copy(y_hbm.at[gidx_v.at[nk]], rows[b],
                                     g_sems[b])

    plsc.subcore_barrier()
    pltpu.sync_copy(acc_sh.at[pl.ds(row0, ROWS_PER_TILE)],
                    out_hbm.at[c, pl.ds(row0, ROWS_PER_TILE)])


def _prep_body(src_ref, type_ref, g_ref):
    g_ref[...] = type_ref[...] * N_NODES + src_ref[...]


_prep = pl.pallas_call(
    _prep_body,
    out_shape=jax.ShapeDtypeStruct((E_EDGES // 128, 128), jnp.int32),
)


_BA = 2000


def _add_body(a_ref, b_ref, o_ref):
    o_ref[...] = a_ref[...] + b_ref[...]


_add = pl.pallas_call(
    _add_body,
    grid=(N_NODES // _BA,),

    in_specs=[
        pl.BlockSpec((_BA, D), lambda i: (i, 0)),
        pl.BlockSpec((_BA, D), lambda i: (i, 0)),
    ],
    out_specs=pl.BlockSpec((_BA, D), lambda i: (i, 0)),
    out_shape=jax.ShapeDtypeStruct((N_NODES, D), jnp.float32),
)


def kernel(x, edge_index, edge_type, edge_norm, weights):
    src2 = edge_index[0].reshape(E_EDGES // 128, 128)
    type2 = edge_type.reshape(E_EDGES // 128, 128)
    gidx = _prep(src2, type2).reshape(NW, NBLOCK, BPC, CHUNK)
    dst4 = edge_index[1].reshape(NW, NBLOCK, BPC, CHUNK)
    norm3 = edge_norm.reshape(NW, NBLOCK, BPC * CHUNK)
    y = _mm(x, weights).reshape(R * N_NODES, D)
    partial = _sc_edges(y, gidx, dst4, norm3)
    return _add(partial[0], partial[1])


# speedup vs baseline: 6.6118x; 1.0391x over previous
import dataclasses
import functools

import jax
import jax.numpy as jnp
from jax import lax
from jax.experimental import pallas as pl
from jax.experimental.pallas import tpu as pltpu
from jax.experimental.pallas import tpu_sc as plsc

N_NODES = 10000
E_EDGES = 320000
D = 128
R = 8

NC = 2
NS = 16
NW = NC * NS
LANES = 16

EDGES_PER_TILE = E_EDGES // NW
CHUNK = 40
BPC = 50
NBLOCK = EDGES_PER_TILE // (BPC * CHUNK)
N_PAD = 10240
ROWS_PER_TILE = N_PAD // NS


_BN = 2000


def _mm_body(x_ref, w_ref, y_ref):
    y_ref[0] = lax.dot_general(
        x_ref[...], w_ref[0], (((1,), (0,)), ((), ())),
        preferred_element_type=jnp.float32,
        precision=lax.Precision.DEFAULT,
    )


_mm = pl.pallas_call(
    _mm_body,
    grid=(R, N_NODES // _BN),
    in_specs=[
        pl.BlockSpec((_BN, D), lambda r, nb: (nb, 0)),
        pl.BlockSpec((1, D, D), lambda r, nb: (r, 0, 0)),
    ],
    out_specs=pl.BlockSpec((1, _BN, D), lambda r, nb: (r, nb, 0)),
    out_shape=jax.ShapeDtypeStruct((R, N_NODES, D), jnp.float32),
)


_mesh = plsc.VectorSubcoreMesh(core_axis_name="c", subcore_axis_name="s")

_sc_params = pltpu.CompilerParams()
if "needs_layout_passes" in pltpu.CompilerParams.__dataclass_fields__:
    _sc_params = dataclasses.replace(_sc_params, needs_layout_passes=False)


NBUF = 2


@functools.partial(
    pl.kernel,
    out_type=jax.ShapeDtypeStruct((NC, N_PAD, D), jnp.float32),
    mesh=_mesh,
    compiler_params=_sc_params,
    scratch_types=[
        pltpu.VMEM((BPC, CHUNK), jnp.int32),
        pltpu.VMEM((BPC, CHUNK), jnp.int32),
        pltpu.VMEM((BPC * CHUNK,), jnp.float32),
        *[pltpu.VMEM((CHUNK, D), jnp.float32) for _ in range(NBUF)],
        pltpu.VMEM_SHARED((N_PAD, D), jnp.float32),
        *[pltpu.SemaphoreType.DMA for _ in range(2 * NBUF)],
    ],
)
def _sc_edges(y_hbm, gidx_hbm, dst_hbm, norm_hbm, out_hbm,
              gidx_v, dst_v, norm_v, *bufs_and_sems):
    rows = bufs_and_sems[:NBUF]
    acc_sh = bufs_and_sems[NBUF]
    g_sems = bufs_and_sems[NBUF + 1:NBUF + 1 + NBUF]
    s_sems = bufs_and_sems[NBUF + 1 + NBUF:]
    c = lax.axis_index("c")
    s = lax.axis_index("s")
    wid = c * NS + s

    zero16 = jnp.zeros((LANES,), jnp.float32)

    @pl.loop(0, CHUNK)
    def _zero_rows(i):
        for j in range(D // LANES):
            rows[0][i, pl.ds(j * LANES, LANES)] = zero16

    row0 = s * ROWS_PER_TILE
    for i in range(ROWS_PER_TILE // CHUNK):
        pltpu.sync_copy(rows[0].at[pl.ds(0, CHUNK)],
                        acc_sh.at[pl.ds(row0 + i * CHUNK, CHUNK)])
    plsc.subcore_barrier()

    @pl.loop(0, NBLOCK)
    def _block(blk):
        pltpu.sync_copy(gidx_hbm.at[wid, blk], gidx_v)
        pltpu.sync_copy(dst_hbm.at[wid, blk], dst_v)
        pltpu.sync_copy(norm_hbm.at[wid, blk], norm_v)

        for b in range(NBUF):
            pltpu.async_copy(y_hbm.at[gidx_v.at[b]], rows[b], g_sems[b])

        @pl.loop(0, BPC, step=NBUF)
        def _group(k0):
            for b in range(NBUF):
                k = k0 + b
                pltpu.make_async_copy(y_hbm.at[gidx_v.at[0]], rows[b],
                                      g_sems[b]).wait()

                @plsc.parallel_loop(0, CHUNK, unroll=2)
                def _scale(e):
                    nvec = plsc.load_gather(
                        norm_v,
                        [jnp.zeros((LANES,), jnp.int32) + (k * CHUNK + e)])
                    for j in range(D // LANES):
                        sl = pl.ds(j * LANES, LANES)
                        rows[b][e, sl] = rows[b][e, sl] * nvec

                pltpu.async_copy(rows[b], acc_sh.at[dst_v.at[k]], s_sems[b],
                                 add=True)
            for b in range(NBUF):
                pltpu.make_async_copy(rows[b], acc_sh.at[dst_v.at[0]],
                                      s_sems[b]).wait()
                nk = k0 + NBUF + b

                @pl.when(nk < BPC)
                def _prefetch():
                    pltpu.async_copy(y_hbm.at[gidx_v.at[nk]], rows[b],
                                     g_sems[b])

    plsc.subcore_barrier()
    pltpu.sync_copy(acc_sh.at[pl.ds(row0, ROWS_PER_TILE)],
                    out_hbm.at[c, pl.ds(row0, ROWS_PER_TILE)])


def _prep_body(src_ref, type_ref, g_ref):
    g_ref[...] = type_ref[...] * N_NODES + src_ref[...]


_prep = pl.pallas_call(
    _prep_body,
    out_shape=jax.ShapeDtypeStruct((E_EDGES // 128, 128), jnp.int32),
)


_BA = 2000


def _add_body(a_ref, b_ref, o_ref):
    o_ref[...] = a_ref[...] + b_ref[...]


_add = pl.pallas_call(
    _add_body,
    grid=(N_NODES // _BA,),

    in_specs=[
        pl.BlockSpec((_BA, D), lambda i: (i, 0)),
        pl.BlockSpec((_BA, D), lambda i: (i, 0)),
    ],
    out_specs=pl.BlockSpec((_BA, D), lambda i: (i, 0)),
    out_shape=jax.ShapeDtypeStruct((N_NODES, D), jnp.float32),
)


def kernel(x, edge_index, edge_type, edge_norm, weights):
    src2 = edge_index[0].reshape(E_EDGES // 128, 128)
    type2 = edge_type.reshape(E_EDGES // 128, 128)
    gidx = _prep(src2, type2).reshape(NW, NBLOCK, BPC, CHUNK)
    dst4 = edge_index[1].reshape(NW, NBLOCK, BPC, CHUNK)
    norm3 = edge_norm.reshape(NW, NBLOCK, BPC * CHUNK)
    y = _mm(x, weights).reshape(R * N_NODES, D)
    partial = _sc_edges(y, gidx, dst4, norm3)
    return _add(partial[0], partial[1])
